# baseline (speedup 1.0000x reference)
import functools
import math

import jax
import jax.numpy as jnp
from jax import lax
from jax.experimental import pallas as pl
from jax.experimental.pallas import tpu as pltpu
from jax.experimental.pallas import tpu_sc as plsc

N = 2048
HID = 32
DEPTH = 3
HI = jax.lax.Precision.HIGHEST


_EB = 32768
_RB = 32
_BLK = _RB * N


def _sc_adj_kernel(ef_hbm, adj_hbm, edges_v, acc):
    wid = lax.axis_index("s") * 2 + lax.axis_index("c")
    pltpu.sync_copy(ef_hbm, edges_v)
    ones = jnp.ones((16,), jnp.float32)
    quarters = jnp.full((16,), 0.25, jnp.float32)
    lane = lax.iota(jnp.int32, 16)
    expect = (jnp.where(lane < 8, 0.5, 0.0)
              + jnp.where(lane < 4, 1.0, 0.0)
              + jnp.where(lane < 1, 4.0, 0.0))

    for b in range(2):
        base = (wid * 2 + b) * _BLK

        def zbody(i, _):
            for q in range(8):
                acc[pl.ds(i * 128 + q * 16, 16)] = jnp.zeros((16,), jnp.float32)
            return 0

        lax.fori_loop(0, _BLK // 128, zbody, 0)

        plsc.addupdate_scatter(acc, [lane >> 1], quarters)
        plsc.addupdate_scatter(acc, [lane >> 2], quarters)
        plsc.addupdate_scatter(acc, [lane & 0], quarters)
        acc[pl.ds(0, 16)] = acc[pl.ds(0, 16)] - expect

        def ebody(i, _):
            v = edges_v[pl.ds(i * 16, 16)]
            m = (v >= base) & (v < base + _BLK)
            local = jnp.where(m, v - base, lane)
            plsc.addupdate_scatter(acc, [local], ones, mask=m)
            return 0

        lax.fori_loop(0, _EB // 16, ebody, 0)
        pltpu.sync_copy(acc, adj_hbm.at[pl.ds(base, _BLK)])


def _sc_build_adj(ef):
    mesh = plsc.VectorSubcoreMesh(core_axis_name="c", subcore_axis_name="s")
    k = functools.partial(
        pl.kernel,
        mesh=mesh,
        compiler_params=pltpu.CompilerParams(needs_layout_passes=False),
        out_type=jax.ShapeDtypeStruct((N * N,), jnp.float32),
        scratch_types=[
            pltpu.VMEM((_EB,), jnp.int32),
            pltpu.VMEM((_BLK,), jnp.float32),
        ],
    )(_sc_adj_kernel)
    return k(ef).reshape(N, N)


def _bf16_parts(m, parts):
    out = []
    r = m
    for _ in range(parts - 1):
        h = r.astype(jnp.bfloat16)
        out.append(h)
        r = r - h.astype(jnp.float32)
    out.append(r.astype(jnp.bfloat16))
    return out


def _augment2_body(a_ref, b_ref, out_ref, *, bm, bn, nk):
    i = pl.program_id(0)
    j = pl.program_id(1)
    a = a_ref[...]
    b = b_ref[...]
    mm = jnp.dot(a.astype(jnp.bfloat16), b.astype(jnp.bfloat16),
                 preferred_element_type=jnp.float32)
    sub = a_ref[:, pl.ds(j * bn, bn)] if nk != bn else a
    res = mm + 2.0 * sub
    ri = lax.broadcasted_iota(jnp.int32, (bm, bn), 0) + i * bm
    ci = lax.broadcasted_iota(jnp.int32, (bm, bn), 1) + j * bn
    out_ref[...] = jnp.where(ri == ci, 0.0, res)


def _tc_augment(adj):
    n = adj.shape[0]
    if n == N:
        bm = bn = 512
        grid = (n // bm, n // bn)
    else:
        bm = bn = n
        grid = (1, 1)
    return pl.pallas_call(
        functools.partial(_augment2_body, bm=bm, bn=bn, nk=n),
        grid=grid,
        in_specs=[
            pl.BlockSpec((bm, n), lambda i, j: (i, 0)),
            pl.BlockSpec((n, bn), lambda i, j: (0, j)),
        ],
        out_specs=pl.BlockSpec((bm, bn), lambda i, j: (i, j)),
        out_shape=jax.ShapeDtypeStruct((n, n), jnp.float32),
    )(adj, adj)


def _gcn_body(adj_ref, x_ref, w_ref, b_ref, out_ref, *, relu, sig):
    adj = adj_ref[...]
    deg = jnp.sum(adj, axis=0) + 2.0
    dis = lax.rsqrt(deg)
    zz = jnp.dot(x_ref[...], w_ref[...], precision=HI)
    zs = zz * dis[:, None]
    t = lax.dot_general(adj, zs, (((0,), (0,)), ((), ())), precision=HI)
    o = (t + 2.0 * zs) * dis[:, None] + b_ref[...]
    if relu:
        o = jnp.maximum(o, 0.0)
    if sig:
        o = jax.nn.sigmoid(o)
    out_ref[...] = o


def _tc_gcn(adj, x, W, b, relu):
    n = adj.shape[0]
    return pl.pallas_call(
        functools.partial(_gcn_body, relu=relu, sig=False),
        out_shape=jax.ShapeDtypeStruct((n, W.shape[1]), jnp.float32),
    )(adj, x, W, b[None, :])


def _gcn_up_body(adj_ref, res_ref, xd_ref, perm_ref, w_ref, b_ref, out_ref,
                 *, relu, sig, n, k):
    up_sel = (perm_ref[...][None, :] ==
              lax.broadcasted_iota(jnp.int32, (n, k), 0)).astype(jnp.float32)
    up = jnp.dot(up_sel, xd_ref[...], precision=HI)
    x = res_ref[...] + up
    adj = adj_ref[...]
    deg = jnp.sum(adj, axis=0) + 2.0
    dis = lax.rsqrt(deg)
    zz = jnp.dot(x, w_ref[...], precision=HI)
    zs = zz * dis[:, None]
    t = lax.dot_general(adj, zs, (((0,), (0,)), ((), ())), precision=HI)
    o = (t + 2.0 * zs) * dis[:, None] + b_ref[...]
    if relu:
        o = jnp.maximum(o, 0.0)
    if sig:
        o = jax.nn.sigmoid(o)
    out_ref[...] = o


def _tc_gcn_up(adj, res, xd, perm, W, b, relu, sig):
    n = adj.shape[0]
    k = xd.shape[0]
    return pl.pallas_call(
        functools.partial(_gcn_up_body, relu=relu, sig=sig, n=n, k=k),
        out_shape=jax.ShapeDtypeStruct((n, W.shape[1]), jnp.float32),
    )(adj, res, xd, perm, W, b[None, :])


def _before(v, i, pv, pi):
    return (v > pv) | ((v == pv) & (i < pi))


def _topk_body(x_ref, w_ref, vs_ref, is_ref, *, n):
    w = w_ref[...]
    nrm = jnp.sqrt(jnp.sum(w * w))
    s = jnp.sum(x_ref[...] * w, axis=1) / nrm
    v = jnp.tanh(s) + 0.0
    idx = lax.iota(jnp.int32, n)
    i_ = idx
    kk = 2
    while kk <= n:
        j = kk // 2
        while j >= 1:
            pv = jnp.where((i_ & j) == 0, jnp.roll(v, -j), jnp.roll(v, j))
            pi = jnp.where((i_ & j) == 0, jnp.roll(idx, -j), jnp.roll(idx, j))
            up = (i_ & kk) == 0
            lower = (i_ & j) == 0
            take_self = jnp.logical_xor(_before(v, idx, pv, pi),
                                        jnp.logical_xor(lower, up))
            take_self = jnp.logical_not(take_self)
            v = jnp.where(take_self, pv, v)
            idx = jnp.where(take_self, pi, idx)
            j //= 2
        kk *= 2
    vs_ref[...] = v
    is_ref[...] = idx


def _tc_score_topk(x, w):
    n = x.shape[0]
    return pl.pallas_call(
        functools.partial(_topk_body, n=n),
        out_shape=(jax.ShapeDtypeStruct((n,), jnp.float32),
                   jax.ShapeDtypeStruct((n,), jnp.int32)),
    )(x, w[None, :])


def _permute_body(a2_ref, perm_ref, x_ref, sv_ref, adj_ref, xp_ref,
                  *, n, k, parts):
    perm = perm_ref[...]
    p_kn = (perm[:, None] ==
            lax.broadcasted_iota(jnp.int32, (k, n), 1))
    pt_nk = (perm[None, :] ==
             lax.broadcasted_iota(jnp.int32, (n, k), 0))
    p16 = p_kn.astype(jnp.bfloat16)
    pt16 = pt_nk.astype(jnp.bfloat16)
    acc = jnp.zeros((k, k), jnp.float32)
    for part in _bf16_parts(a2_ref[...], parts):
        t1 = jnp.dot(part, pt16, preferred_element_type=jnp.float32)
        acc = acc + jnp.dot(p16, t1.astype(jnp.bfloat16),
                            preferred_element_type=jnp.float32)
    adj_ref[...] = acc
    xp = jnp.dot(p_kn.astype(jnp.float32), x_ref[...], precision=HI)
    xp_ref[...] = xp * sv_ref[...][:, None]


def _tc_permute(A2, perm, x, sval, parts):
    n = A2.shape[0]
    k = perm.shape[0]
    return pl.pallas_call(
        functools.partial(_permute_body, n=n, k=k, parts=parts),
        out_shape=(jax.ShapeDtypeStruct((k, k), jnp.float32),
                   jax.ShapeDtypeStruct((k, HID), jnp.float32)),
    )(A2, perm, x, sval)


_PARTS = {1: 1, 2: 2, 3: 3}


def _unet(p, x, adj, A2_0):
    x0 = _tc_gcn(adj, x, p["down_W"][0], p["down_b"][0], relu=True)
    xs, adjs, perms = [x0], [adj], []
    xcur, curadj = x0, adj
    for i in range(1, DEPTH + 1):
        A2 = A2_0 if i == 1 else _tc_augment(curadj)
        vals, idxs = _tc_score_topk(xcur, p["pool_w"][i - 1])
        k = xcur.shape[0] // 2
        sval, perm = vals[:k], idxs[:k]
        adj_l, xp = _tc_permute(A2, perm, xcur, sval, _PARTS[i])
        xcur = _tc_gcn(adj_l, xp, p["down_W"][i], p["down_b"][i], relu=True)
        curadj = adj_l
        if i < DEPTH:
            xs.append(xcur)
            adjs.append(adj_l)
        perms.append(perm)
    for i in range(DEPTH):
        j = DEPTH - 1 - i
        xcur = _tc_gcn_up(adjs[j], xs[j], xcur, perms[j],
                          p["up_W"][i], p["up_b"][i],
                          relu=(i < DEPTH - 1), sig=(i == DEPTH - 1))
    return xcur


def kernel(x, edge_index, params_A, params_V):
    src = edge_index[0].astype(jnp.int32)
    dst = edge_index[1].astype(jnp.int32)
    ef = src * N + dst
    adj = _sc_build_adj(ef)
    A2_0 = _tc_augment(adj)
    out_A = _unet(params_A, x, adj, A2_0)
    out_V = _unet(params_V, x, adj, A2_0)
    return (out_A, out_V)
